# baseline (speedup 1.0000x reference)
import functools

import jax
import jax.numpy as jnp
from jax import lax
from jax.experimental import pallas as pl
from jax.experimental.pallas import tpu as pltpu

_ALPHA = 0.2
_NEG = -9e15
_NT = (((1,), (1,)), ((), ()))


def _gat_block_kernel(x_ref, adj_ref, w_ref, b_ref, a_ref, out_ref,
                      nf_scr, sdst_scr, mean_scr, smax_scr, epre_scr, *,
                      block_rows, c_out):
    i = pl.program_id(0)
    n = nf_scr.shape[0]

    def unmasked_e(nf_rows, sdst_row, smax):
        s_src = lax.dot_general(nf_rows, a_ref[:, :c_out], _NT,
                                preferred_element_type=jnp.float32)
        t = s_src + smax
        m = jnp.maximum(t, _ALPHA * t)
        logits = s_src + sdst_row
        logits = jnp.maximum(logits, _ALPHA * logits)
        return jnp.exp(logits - m)

    @pl.when(i == 0)
    def _init():
        nf = lax.dot_general(x_ref[...], w_ref[...], _NT,
                             preferred_element_type=jnp.float32)
        nf = nf + b_ref[...]
        nf_scr[:, :c_out] = nf
        nf_scr[:, c_out:] = jnp.ones((n, 1), jnp.float32)
        sdst = lax.dot_general(a_ref[:, c_out:], nf, _NT,
                               preferred_element_type=jnp.float32)
        sdst_scr[...] = sdst
        smax_scr[0, 0] = jnp.max(sdst)
        mean_scr[...] = jnp.sum(nf, axis=0, keepdims=True) / n
        epre_scr[...] = unmasked_e(nf[n - block_rows:, :], sdst,
                                   smax_scr[0, 0])

    @pl.when(i < pl.num_programs(0) - 1)
    def _body():
        nf_rows = nf_scr[pl.ds(i * block_rows, block_rows), :c_out]
        e = jnp.where(adj_ref[...] == 1,
                      unmasked_e(nf_rows, sdst_scr[...], smax_scr[0, 0]),
                      0.0)
        agg = jnp.dot(e, nf_scr[...], preferred_element_type=jnp.float32)
        s = agg[:, c_out:]
        out_ref[...] = jnp.where(s == 0.0, mean_scr[...],
                                 agg[:, :c_out] / s)

    @pl.when(i == pl.num_programs(0) - 1)
    def _tail():
        e = jnp.where(adj_ref[...] == 1, epre_scr[...], 0.0)
        agg = jnp.dot(e, nf_scr[...], preferred_element_type=jnp.float32)
        s = agg[:, c_out:]
        out_ref[...] = jnp.where(s == 0.0, mean_scr[...],
                                 agg[:, :c_out] / s)


@jax.jit
def kernel(node_feats, adj_matrix, W, b, a):
    batch, n, c_in = node_feats.shape
    c_out = W.shape[0]
    x = node_feats.reshape(n, c_in)
    adj = adj_matrix.reshape(n, n)
    b_row = b.reshape(1, c_out)

    block_rows = 512
    grid = n // block_rows

    out = pl.pallas_call(
        functools.partial(_gat_block_kernel, block_rows=block_rows,
                          c_out=c_out),
        grid=(grid,),
        in_specs=[
            pl.BlockSpec((n, c_in), lambda i: (0, 0)),
            pl.BlockSpec((block_rows, n), lambda i: (i, 0)),
            pl.BlockSpec((c_out, c_in), lambda i: (0, 0)),
            pl.BlockSpec((1, c_out), lambda i: (0, 0)),
            pl.BlockSpec((1, 2 * c_out), lambda i: (0, 0)),
        ],
        out_specs=pl.BlockSpec((block_rows, c_out), lambda i: (i, 0)),
        scratch_shapes=[
            pltpu.VMEM((n, c_out + 1), jnp.float32),
            pltpu.VMEM((1, n), jnp.float32),
            pltpu.VMEM((1, c_out), jnp.float32),
            pltpu.SMEM((1, 1), jnp.float32),
            pltpu.VMEM((block_rows, n), jnp.float32),
        ],
        out_shape=jax.ShapeDtypeStruct((n, c_out), jnp.float32),
        compiler_params=pltpu.CompilerParams(
            dimension_semantics=("arbitrary",),
        ),
    )(x, adj, W, b_row, a)

    return out.reshape(batch, n, c_out)


# speedup vs baseline: 1.0217x; 1.0217x over previous
import functools

import jax
import jax.numpy as jnp
from jax import lax
from jax.experimental import pallas as pl
from jax.experimental.pallas import tpu as pltpu

_ALPHA = 0.2
_NT = (((1,), (1,)), ((), ()))


def _gat_block_kernel(x_ref, adj_ref, w_ref, b_ref, a_ref, out_ref,
                      nf_scr, sdst_scr, mean_scr, smax_scr, *,
                      block_rows, c_out):
    i = pl.program_id(0)
    n = nf_scr.shape[0]

    @pl.when(i == 0)
    def _init():
        nf = lax.dot_general(x_ref[...], w_ref[...], _NT,
                             preferred_element_type=jnp.float32)
        nf = nf + b_ref[...]
        nf_scr[:, :c_out] = nf
        nf_scr[:, c_out:] = jnp.ones((n, 1), jnp.float32)
        sdst = lax.dot_general(a_ref[:, c_out:], nf, _NT,
                               preferred_element_type=jnp.float32)
        sdst_scr[...] = sdst
        smax_scr[0, 0] = jnp.max(sdst)
        mean_scr[...] = jnp.sum(nf, axis=0, keepdims=True) / n

    nf_rows = nf_scr[pl.ds(i * block_rows, block_rows), :c_out]
    s_src = lax.dot_general(nf_rows, a_ref[:, :c_out], _NT,
                            preferred_element_type=jnp.float32)
    t = s_src + smax_scr[0, 0]
    m = jnp.maximum(t, _ALPHA * t)
    logits = s_src + sdst_scr[...]
    logits = jnp.maximum(logits, _ALPHA * logits)
    e = jnp.where(adj_ref[...] == 1, jnp.exp(logits - m), 0.0)
    agg = jnp.dot(e, nf_scr[...], preferred_element_type=jnp.float32)
    s = agg[:, c_out:]
    out_ref[...] = jnp.where(s == 0.0, mean_scr[...], agg[:, :c_out] / s)


@jax.jit
def kernel(node_feats, adj_matrix, W, b, a):
    batch, n, c_in = node_feats.shape
    c_out = W.shape[0]
    x = node_feats.reshape(n, c_in)
    adj = adj_matrix.reshape(n, n)
    b_row = b.reshape(1, c_out)

    block_rows = 512
    grid = n // block_rows

    out = pl.pallas_call(
        functools.partial(_gat_block_kernel, block_rows=block_rows,
                          c_out=c_out),
        grid=(grid,),
        in_specs=[
            pl.BlockSpec((n, c_in), lambda i: (0, 0)),
            pl.BlockSpec((block_rows, n), lambda i: (i, 0)),
            pl.BlockSpec((c_out, c_in), lambda i: (0, 0)),
            pl.BlockSpec((1, c_out), lambda i: (0, 0)),
            pl.BlockSpec((1, 2 * c_out), lambda i: (0, 0)),
        ],
        out_specs=pl.BlockSpec((block_rows, c_out), lambda i: (i, 0)),
        scratch_shapes=[
            pltpu.VMEM((n, c_out + 1), jnp.float32),
            pltpu.VMEM((1, n), jnp.float32),
            pltpu.VMEM((1, c_out), jnp.float32),
            pltpu.SMEM((1, 1), jnp.float32),
        ],
        out_shape=jax.ShapeDtypeStruct((n, c_out), jnp.float32),
        compiler_params=pltpu.CompilerParams(
            dimension_semantics=("arbitrary",),
        ),
    )(x, adj, W, b_row, a)

    return out.reshape(batch, n, c_out)


# speedup vs baseline: 1.0416x; 1.0195x over previous
import functools

import jax
import jax.numpy as jnp
from jax import lax
from jax.experimental import pallas as pl
from jax.experimental.pallas import tpu as pltpu

_ALPHA = 0.2
_NT = (((1,), (1,)), ((), ()))


def _gat_block_kernel(x_ref, adj_ref, w_ref, b_ref, a_ref, out_ref,
                      nf_scr, sdst_scr, mean_scr, smax_scr, *,
                      block_rows, c_out):
    i = pl.program_id(0)
    n = nf_scr.shape[0]

    @pl.when(i == 0)
    def _init():
        nf = lax.dot_general(x_ref[...], w_ref[...], _NT,
                             preferred_element_type=jnp.float32)
        nf = nf + b_ref[...]
        nf_scr[:, :c_out] = nf
        nf_scr[:, c_out:] = jnp.ones((n, 1), jnp.float32)
        sdst = lax.dot_general(a_ref[:, c_out:], nf, _NT,
                               preferred_element_type=jnp.float32)
        sdst_scr[0:1, :] = sdst
        sdst_scr[1:2, :] = _ALPHA * sdst
        smax_scr[0, 0] = jnp.max(sdst)
        mean_scr[...] = jnp.sum(nf, axis=0, keepdims=True) / n

    nf_rows = nf_scr[pl.ds(i * block_rows, block_rows), :c_out]
    s_src = lax.dot_general(nf_rows, a_ref[:, :c_out], _NT,
                            preferred_element_type=jnp.float32)
    t = s_src + smax_scr[0, 0]
    m = jnp.maximum(t, _ALPHA * t)
    q1 = s_src - m
    q2 = _ALPHA * s_src - m
    e_arg = jnp.maximum(q1 + sdst_scr[0:1, :], q2 + sdst_scr[1:2, :])
    e = jnp.where(adj_ref[...] == 1, jnp.exp(e_arg), 0.0)
    agg = jnp.dot(e, nf_scr[...], preferred_element_type=jnp.float32)
    s = agg[:, c_out:]
    out_ref[...] = jnp.where(s == 0.0, mean_scr[...], agg[:, :c_out] / s)


@jax.jit
def kernel(node_feats, adj_matrix, W, b, a):
    batch, n, c_in = node_feats.shape
    c_out = W.shape[0]
    x = node_feats.reshape(n, c_in)
    adj = adj_matrix.reshape(n, n)
    b_row = b.reshape(1, c_out)

    block_rows = 512
    grid = n // block_rows

    out = pl.pallas_call(
        functools.partial(_gat_block_kernel, block_rows=block_rows,
                          c_out=c_out),
        grid=(grid,),
        in_specs=[
            pl.BlockSpec((n, c_in), lambda i: (0, 0)),
            pl.BlockSpec((block_rows, n), lambda i: (i, 0)),
            pl.BlockSpec((c_out, c_in), lambda i: (0, 0)),
            pl.BlockSpec((1, c_out), lambda i: (0, 0)),
            pl.BlockSpec((1, 2 * c_out), lambda i: (0, 0)),
        ],
        out_specs=pl.BlockSpec((block_rows, c_out), lambda i: (i, 0)),
        scratch_shapes=[
            pltpu.VMEM((n, c_out + 1), jnp.float32),
            pltpu.VMEM((2, n), jnp.float32),
            pltpu.VMEM((1, c_out), jnp.float32),
            pltpu.SMEM((1, 1), jnp.float32),
        ],
        out_shape=jax.ShapeDtypeStruct((n, c_out), jnp.float32),
        compiler_params=pltpu.CompilerParams(
            dimension_semantics=("arbitrary",),
        ),
    )(x, adj, W, b_row, a)

    return out.reshape(batch, n, c_out)


# speedup vs baseline: 1.0455x; 1.0038x over previous
import functools

import jax
import jax.numpy as jnp
from jax import lax
from jax.experimental import pallas as pl
from jax.experimental.pallas import tpu as pltpu

_ALPHA = 0.2
_LOG2E = 1.4426950408889634
_NT = (((1,), (1,)), ((), ()))


def _gat_block_kernel(x_ref, adj_ref, w_ref, b_ref, a_ref, out_ref,
                      nf_scr, sdst_scr, mean_scr, smax_scr, *,
                      block_rows, c_out):
    i = pl.program_id(0)
    n = nf_scr.shape[0]

    @pl.when(i == 0)
    def _init():
        nf = lax.dot_general(x_ref[...], w_ref[...], _NT,
                             preferred_element_type=jnp.float32)
        nf = nf + b_ref[...]
        nf_scr[:, :c_out] = nf
        nf_scr[:, c_out:] = jnp.ones((n, 1), jnp.float32)
        sdst = lax.dot_general(a_ref[:, c_out:], nf, _NT,
                               preferred_element_type=jnp.float32)
        sdst_scr[0:1, :] = _LOG2E * sdst
        sdst_scr[1:2, :] = (_ALPHA * _LOG2E) * sdst
        smax_scr[0, 0] = jnp.max(sdst)
        mean_scr[...] = jnp.sum(nf, axis=0, keepdims=True) / n

    nf_rows = nf_scr[pl.ds(i * block_rows, block_rows), :c_out]
    s_src = lax.dot_general(nf_rows, a_ref[:, :c_out], _NT,
                            preferred_element_type=jnp.float32)
    t = s_src + smax_scr[0, 0]
    m = jnp.maximum(t, _ALPHA * t)
    q1 = _LOG2E * (s_src - m)
    q2 = _LOG2E * (_ALPHA * s_src - m)
    e_arg = jnp.maximum(q1 + sdst_scr[0:1, :], q2 + sdst_scr[1:2, :])
    e = jnp.where(adj_ref[...] == 1, jnp.exp2(e_arg), 0.0)
    agg = jnp.dot(e, nf_scr[...], preferred_element_type=jnp.float32)
    s = agg[:, c_out:]
    out_ref[...] = jnp.where(s == 0.0, mean_scr[...], agg[:, :c_out] / s)


@jax.jit
def kernel(node_feats, adj_matrix, W, b, a):
    batch, n, c_in = node_feats.shape
    c_out = W.shape[0]
    x = node_feats.reshape(n, c_in)
    adj = adj_matrix.reshape(n, n)
    b_row = b.reshape(1, c_out)

    block_rows = 512
    grid = n // block_rows

    out = pl.pallas_call(
        functools.partial(_gat_block_kernel, block_rows=block_rows,
                          c_out=c_out),
        grid=(grid,),
        in_specs=[
            pl.BlockSpec((n, c_in), lambda i: (0, 0)),
            pl.BlockSpec((block_rows, n), lambda i: (i, 0)),
            pl.BlockSpec((c_out, c_in), lambda i: (0, 0)),
            pl.BlockSpec((1, c_out), lambda i: (0, 0)),
            pl.BlockSpec((1, 2 * c_out), lambda i: (0, 0)),
        ],
        out_specs=pl.BlockSpec((block_rows, c_out), lambda i: (i, 0)),
        scratch_shapes=[
            pltpu.VMEM((n, c_out + 1), jnp.float32),
            pltpu.VMEM((2, n), jnp.float32),
            pltpu.VMEM((1, c_out), jnp.float32),
            pltpu.SMEM((1, 1), jnp.float32),
        ],
        out_shape=jax.ShapeDtypeStruct((n, c_out), jnp.float32),
        compiler_params=pltpu.CompilerParams(
            dimension_semantics=("arbitrary",),
        ),
    )(x, adj, W, b_row, a)

    return out.reshape(batch, n, c_out)
